# baseline (device time: 13016 ns/iter reference)
import jax
import jax.numpy as jnp
from jax import lax
from jax.experimental import pallas as pl
from jax.experimental.pallas import tpu as pltpu

NQ = 8


def kernel(x):
    m, n = x.shape
    h = m // NQ

    def body(x_ref, out_ref, comm_ref, send_sems, recv_sems):
        my_x = lax.axis_index("x")
        my_y = lax.axis_index("y")
        y_nbr = (my_x, 1 - my_y)
        x_nbr = (1 - my_x, my_y)

        barrier_sem = pltpu.get_barrier_semaphore()
        for nbr in (y_nbr, x_nbr):
            pl.semaphore_signal(
                barrier_sem, inc=1,
                device_id=nbr, device_id_type=pl.DeviceIdType.MESH,
            )
        pl.semaphore_wait(barrier_sem, 2)

        qs = []
        for k in range(NQ // 2):
            qs.append((k, y_nbr, x_nbr))
            qs.append((NQ // 2 + k, x_nbr, y_nbr))

        p1 = {}
        for q, nbr1, _ in qs:
            comm_ref[q, :, :] = x_ref[q * h:(q + 1) * h, :].astype(jnp.bfloat16)
            r = pltpu.make_async_remote_copy(
                src_ref=comm_ref.at[q], dst_ref=comm_ref.at[4 + q],
                send_sem=send_sems.at[q], recv_sem=recv_sems.at[q],
                device_id=nbr1, device_id_type=pl.DeviceIdType.MESH,
            )
            r.start()
            p1[q] = r

        p2 = {}
        for q, _, nbr2 in qs:
            p1[q].wait_recv()
            comm_ref[8 + q, :, :] = comm_ref[q, :, :] + comm_ref[4 + q, :, :]
            r = pltpu.make_async_remote_copy(
                src_ref=comm_ref.at[8 + q], dst_ref=comm_ref.at[12 + q],
                send_sem=send_sems.at[NQ + q], recv_sem=recv_sems.at[NQ + q],
                device_id=nbr2, device_id_type=pl.DeviceIdType.MESH,
            )
            r.start()
            p2[q] = r

        for q, _, _ in qs:
            p2[q].wait_recv()
            out_ref[q * h:(q + 1) * h, :] = (
                comm_ref[8 + q, :, :] + comm_ref[12 + q, :, :]
            ).astype(jnp.float32)

        for q in range(NQ):
            p1[q].wait_send()
            p2[q].wait_send()

    return pl.pallas_call(
        body,
        out_shape=jax.ShapeDtypeStruct((m, n), jnp.float32),
        in_specs=[pl.BlockSpec(memory_space=pltpu.VMEM)],
        out_specs=pl.BlockSpec(memory_space=pltpu.VMEM),
        scratch_shapes=[
            pltpu.VMEM((4 * NQ, h, n), jnp.bfloat16),
            pltpu.SemaphoreType.DMA((2 * NQ,)),
            pltpu.SemaphoreType.DMA((2 * NQ,)),
        ],
        compiler_params=pltpu.CompilerParams(collective_id=0),
    )(x)


# device time: 12907 ns/iter; 1.0084x vs baseline; 1.0084x over previous
import jax
import jax.numpy as jnp
from jax import lax
from jax.experimental import pallas as pl
from jax.experimental.pallas import tpu as pltpu

NQ = 4


def kernel(x):
    m, n = x.shape
    h = m // NQ

    def body(x_ref, out_ref, comm_ref, send_sems, recv_sems):
        my_x = lax.axis_index("x")
        my_y = lax.axis_index("y")
        y_nbr = (my_x, 1 - my_y)
        x_nbr = (1 - my_x, my_y)

        barrier_sem = pltpu.get_barrier_semaphore()
        for nbr in (y_nbr, x_nbr):
            pl.semaphore_signal(
                barrier_sem, inc=1,
                device_id=nbr, device_id_type=pl.DeviceIdType.MESH,
            )
        pl.semaphore_wait(barrier_sem, 2)

        qs = []
        for k in range(NQ // 2):
            qs.append((k, y_nbr, x_nbr))
            qs.append((NQ // 2 + k, x_nbr, y_nbr))

        p1 = {}
        for q, nbr1, _ in qs:
            comm_ref[q, :, :] = x_ref[q * h:(q + 1) * h, :].astype(jnp.bfloat16)
            r = pltpu.make_async_remote_copy(
                src_ref=comm_ref.at[q], dst_ref=comm_ref.at[NQ + q],
                send_sem=send_sems.at[q], recv_sem=recv_sems.at[q],
                device_id=nbr1, device_id_type=pl.DeviceIdType.MESH,
            )
            r.start()
            p1[q] = r

        p2 = {}
        for q, _, nbr2 in qs:
            p1[q].wait_recv()
            comm_ref[2 * NQ + q, :, :] = (
                comm_ref[q, :, :] + comm_ref[NQ + q, :, :]
            )
            r = pltpu.make_async_remote_copy(
                src_ref=comm_ref.at[2 * NQ + q], dst_ref=comm_ref.at[3 * NQ + q],
                send_sem=send_sems.at[NQ + q], recv_sem=recv_sems.at[NQ + q],
                device_id=nbr2, device_id_type=pl.DeviceIdType.MESH,
            )
            r.start()
            p2[q] = r

        for q, _, _ in qs:
            p2[q].wait_recv()
            out_ref[q * h:(q + 1) * h, :] = (
                comm_ref[2 * NQ + q, :, :] + comm_ref[3 * NQ + q, :, :]
            ).astype(jnp.float32)

        for q in range(NQ):
            p1[q].wait_send()
            p2[q].wait_send()

    return pl.pallas_call(
        body,
        out_shape=jax.ShapeDtypeStruct((m, n), jnp.float32),
        in_specs=[pl.BlockSpec(memory_space=pltpu.VMEM)],
        out_specs=pl.BlockSpec(memory_space=pltpu.VMEM),
        scratch_shapes=[
            pltpu.VMEM((4 * NQ, h, n), jnp.bfloat16),
            pltpu.SemaphoreType.DMA((2 * NQ,)),
            pltpu.SemaphoreType.DMA((2 * NQ,)),
        ],
        compiler_params=pltpu.CompilerParams(collective_id=0),
    )(x)
